# Initial kernel scaffold; baseline (speedup 1.0000x reference)
import jax
import jax.numpy as jnp
from jax.experimental import pallas as pl


def kernel(feature_indices, feature_values, weight, bias):
    raise NotImplementedError("write your pallas kernel here")


# speedup vs baseline: 13.1856x; 13.1856x over previous
import functools

import jax
import jax.numpy as jnp
from jax import lax
from jax.experimental import pallas as pl
from jax.experimental.pallas import tpu as pltpu
from jax.experimental.pallas import tpu_sc as plsc

NUM_INPUTS = 100000
D = 128
B = 16384
K = 50
KPAD = 64

NC = 2
NS = 16
NW = NC * NS
RPW = B // NW
GRP = 2
NG = RPW // GRP
LANES = 16
DCH = D // LANES

_BCAST_DNUMS = lax.GatherDimensionNumbers(
    offset_dims=(), collapsed_slice_dims=(0,), start_index_map=(0,))


def _lane_broadcast(vec, lane):
    idx = jnp.full((LANES, 1), lane, dtype=jnp.int32)
    return lax.gather(vec, idx, _BCAST_DNUMS, (1,),
                      mode=lax.GatherScatterMode.PROMISE_IN_BOUNDS)


def _sc_body(idx_hbm, vals_hbm, weight_hbm, bias_hbm, out_hbm,
             idx_v, vals_v, rows_v, bias_v, out_v, gsem, osem):
    wid = lax.axis_index("s") * NC + lax.axis_index("c")
    row0 = wid * RPW
    grp0 = wid * NG

    pltpu.sync_copy(idx_hbm.at[pl.ds(grp0, NG)], idx_v)
    pltpu.sync_copy(vals_hbm.at[pl.ds(row0 * KPAD, RPW * KPAD)], vals_v)
    pltpu.sync_copy(bias_hbm, bias_v)

    def fire_gather(grp, buf):
        pltpu.async_copy(weight_hbm.at[idx_v.at[grp]], rows_v.at[buf],
                         gsem.at[buf])

    def wait_gather(grp, buf):
        pltpu.make_async_copy(weight_hbm.at[idx_v.at[grp]], rows_v.at[buf],
                              gsem.at[buf]).wait()

    def out_slice(grp):
        return out_hbm.at[pl.ds(row0 + grp * GRP, GRP)]

    fire_gather(0, 0)

    @pl.loop(0, NG, step=2)
    def _grp_loop(g):
        for b in range(2):
            grp = g + b

            @pl.when(grp + 1 < NG)
            def _():
                fire_gather(grp + 1, (b + 1) % 2)

            wait_gather(grp, b)

            @pl.when(g >= 2)
            def _():
                pltpu.make_async_copy(out_v.at[b], out_slice(grp),
                                      osem.at[b]).wait()

            for r in range(GRP):
                rloc = grp * GRP + r
                accs = tuple(bias_v[pl.ds(j * LANES, LANES)]
                             for j in range(DCH))
                for t in range(KPAD // LANES):
                    kcnt = min(LANES, K - t * LANES)
                    if kcnt <= 0:
                        break
                    voff = pl.multiple_of(rloc * KPAD + t * LANES, LANES)
                    vv_t = vals_v[pl.ds(voff, LANES)]

                    @pl.loop(0, kcnt, init_carry=accs, unroll=4)
                    def _k_loop(lane, accs, r=r, b=b, t=t, vv_t=vv_t):
                        vb = _lane_broadcast(vv_t, lane)
                        krow = r * K + t * LANES + lane
                        return tuple(
                            accs[j] + rows_v[b, krow,
                                             pl.ds(j * LANES, LANES)] * vb
                            for j in range(DCH))

                    accs = _k_loop
                for j in range(DCH):
                    out_v[b, r, pl.ds(j * LANES, LANES)] = accs[j]

            pltpu.async_copy(out_v.at[b], out_slice(grp), osem.at[b])

    for b in range(2):
        pltpu.make_async_copy(out_v.at[b], out_hbm.at[pl.ds(row0, GRP)],
                              osem.at[b]).wait()


@jax.jit
def kernel(feature_indices, feature_values, weight, bias):
    idx2 = feature_indices.reshape(B // GRP, GRP * K)
    vals_p = jnp.pad(feature_values, ((0, 0), (0, KPAD - K))).reshape(B * KPAD)

    mesh = plsc.VectorSubcoreMesh(core_axis_name="c", subcore_axis_name="s")
    run = pl.kernel(
        _sc_body,
        out_type=jax.ShapeDtypeStruct((B, D), jnp.float32),
        mesh=mesh,
        scratch_types=[
            pltpu.VMEM((NG, GRP * K), jnp.int32),
            pltpu.VMEM((RPW * KPAD,), jnp.float32),
            pltpu.VMEM((2, GRP * K, D), jnp.float32),
            pltpu.VMEM((D,), jnp.float32),
            pltpu.VMEM((2, GRP, D), jnp.float32),
            pltpu.SemaphoreType.DMA((2,)),
            pltpu.SemaphoreType.DMA((2,)),
        ],
    )
    return run(idx2, vals_p, weight, bias)
